# baseline (speedup 1.0000x reference)
import functools

import jax
import jax.numpy as jnp
from jax import lax
from jax.experimental import pallas as pl
from jax.experimental.pallas import tpu as pltpu
from jax.experimental.pallas import tpu_sc as plsc

B = 16384
D = 16
V = 100000
VE = 1000
NC = 2
L = 16
HB = B // 2
NROW = HB // 128

_mesh = plsc.VectorSubcoreMesh(core_axis_name="c", subcore_axis_name="s")


def _one_feature(idx_hbm, tbl, piece, col_v, idx_v, out_v, sem_a, sem_b,
                 f, h, vocab):
    cc = pltpu.make_async_copy(
        tbl.at[f, pl.ds(0, vocab)], col_v.at[pl.ds(0, vocab)], sem_a
    )
    cc.start()
    ci = pltpu.make_async_copy(
        idx_hbm.at[pl.ds(h * HB, HB)], idx_v.at[...], sem_b
    )
    ci.start()
    cc.wait()
    ci.wait()

    def _blk(g, _):
        for j in range(8):
            iv = idx_v[pl.ds((g * 8 + j) * L, L)]
            out_v[g, pl.ds(j * L, L)] = plsc.load_gather(col_v, [iv])
        return 0

    lax.fori_loop(0, NROW, _blk, 0)

    co = pltpu.make_async_copy(
        out_v.at[...], piece.at[f, pl.ds(h * NROW, NROW)], sem_a
    )
    co.start()
    co.wait()


_scratch = [
    pltpu.VMEM((V,), jnp.float32),
    pltpu.VMEM((HB,), jnp.int32),
    pltpu.VMEM((NROW, 128), jnp.float32),
    pltpu.SemaphoreType.DMA,
    pltpu.SemaphoreType.DMA,
]
_params = pltpu.CompilerParams(
    use_tc_tiling_on_sc=False, needs_layout_passes=False
)


def _make_gather(vocab):
    @functools.partial(
        pl.kernel,
        out_type=jax.ShapeDtypeStruct((D, 128, 128), jnp.float32),
        mesh=_mesh,
        scratch_types=_scratch,
        compiler_params=_params,
    )
    def _gather(idx_hbm, tbl, piece, col_v, idx_v, out_v, sem_a, sem_b):
        wid = lax.axis_index("s") * NC + lax.axis_index("c")
        f = lax.shift_right_logical(wid, 1)
        h = wid & 1
        _one_feature(idx_hbm, tbl, piece, col_v, idx_v, out_v, sem_a, sem_b,
                     f, h, vocab)

    return _gather


_gather_big = _make_gather(V)
_gather_edu = _make_gather(VE)


BLK = 4096


def _mlp_body(p0, p1, p2, p3, w1_ref, b1_ref, w2_ref, b2_ref,
              w3_ref, b3_ref, o_ref):
    x = jnp.concatenate(
        [p0[...].reshape(D, BLK), p1[...].reshape(D, BLK),
         p2[...].reshape(D, BLK), p3[...].reshape(D, BLK)], axis=0
    )
    dn = (((0,), (0,)), ((), ()))
    h = lax.dot_general(w1_ref[...], x, dn,
                        preferred_element_type=jnp.float32)
    h = jnp.maximum(h + b1_ref[...], 0.0)
    h2 = lax.dot_general(w2_ref[...], h, dn,
                         preferred_element_type=jnp.float32)
    h2 = jnp.maximum(h2 + b2_ref[...], 0.0)
    z = lax.dot_general(w3_ref[...], h2, dn,
                        preferred_element_type=jnp.float32)
    o_ref[...] = jax.nn.sigmoid(z + b3_ref[0, 0])[0]


def _mlp(pieces, W1, b1, W2, b2, W3, b3):
    grid = (B // BLK,)
    pspec = pl.BlockSpec((D, BLK // 128, 128), lambda i: (0, i, 0))
    whole = lambda shape: pl.BlockSpec(shape, lambda i: (0,) * len(shape))
    return pl.pallas_call(
        _mlp_body,
        grid=grid,
        in_specs=[
            pspec, pspec, pspec, pspec,
            whole((64, 64)), whole((64, 1)),
            whole((64, 32)), whole((32, 1)),
            whole((32, 1)), whole((1, 1)),
        ],
        out_specs=pl.BlockSpec((BLK,), lambda i: (i,)),
        out_shape=jax.ShapeDtypeStruct((B,), jnp.float32),
    )(*pieces, W1, b1, W2, b2, W3, b3)


def kernel(skills, positions, education, job_position,
           skill_table, position_table, education_table, job_position_table,
           W1, b1, W2, b2, W3, b3):
    sk = skills.astype(jnp.int32)
    po = positions.astype(jnp.int32)
    ed = education.astype(jnp.int32)
    jp = job_position.astype(jnp.int32)

    p_ed = _gather_edu(ed, education_table.T)
    pieces = [
        _gather_big(sk, skill_table.T),
        _gather_big(po, position_table.T),
        p_ed,
        _gather_big(jp, job_position_table.T),
    ]

    return _mlp(
        pieces, W1, b1.reshape(64, 1), W2, b2.reshape(32, 1),
        W3, b3.reshape(1, 1),
    )


# speedup vs baseline: 1.0602x; 1.0602x over previous
import functools

import jax
import jax.numpy as jnp
from jax import lax
from jax.experimental import pallas as pl
from jax.experimental.pallas import tpu as pltpu
from jax.experimental.pallas import tpu_sc as plsc

B = 16384
D = 16
V = 100000
VE = 1000
NC = 2
L = 16
HB = B // 2
NROW = HB // 128

_mesh = plsc.VectorSubcoreMesh(core_axis_name="c", subcore_axis_name="s")


def _one_feature(idx_hbm, tbl, piece, col_v, idx_v, out_v, sem_a, sem_b,
                 f, h, vocab):
    cc = pltpu.make_async_copy(
        tbl.at[f, pl.ds(0, vocab)], col_v.at[pl.ds(0, vocab)], sem_a
    )
    cc.start()
    ci = pltpu.make_async_copy(
        idx_hbm.at[pl.ds(h * HB, HB)], idx_v.at[...], sem_b
    )
    ci.start()
    cc.wait()
    ci.wait()

    def _blk(g, _):
        for j in range(8):
            iv = idx_v[pl.ds((g * 8 + j) * L, L)]
            out_v[g, pl.ds(j * L, L)] = plsc.load_gather(col_v, [iv])
        return 0

    lax.fori_loop(0, NROW, _blk, 0)

    co = pltpu.make_async_copy(
        out_v.at[...], piece.at[f, pl.ds(h * NROW, NROW)], sem_a
    )
    co.start()
    co.wait()


_scratch = [
    pltpu.VMEM((V,), jnp.float32),
    pltpu.VMEM((HB,), jnp.int32),
    pltpu.VMEM((NROW, 128), jnp.float32),
    pltpu.SemaphoreType.DMA,
    pltpu.SemaphoreType.DMA,
]
_params = pltpu.CompilerParams(
    use_tc_tiling_on_sc=False, needs_layout_passes=False
)


@functools.partial(
    pl.kernel,
    out_type=[jax.ShapeDtypeStruct((D, 128, 128), jnp.float32)] * 2,
    mesh=_mesh,
    scratch_types=[
        pltpu.VMEM((V,), jnp.float32),
        pltpu.VMEM((HB,), jnp.int32),
        pltpu.VMEM((NROW, 128), jnp.float32),
        pltpu.VMEM((VE,), jnp.float32),
        pltpu.VMEM((HB,), jnp.int32),
        pltpu.SemaphoreType.DMA,
        pltpu.SemaphoreType.DMA,
        pltpu.SemaphoreType.DMA,
        pltpu.SemaphoreType.DMA,
    ],
    compiler_params=_params,
)
def _gather_big_edu(idx_hbm, eidx_hbm, tbl, etbl, piece, epiece,
                    col_v, idx_v, out_v, ecol_v, eidx_v,
                    sem_a, sem_b, sem_c, sem_d):
    wid = lax.axis_index("s") * NC + lax.axis_index("c")
    f = lax.shift_right_logical(wid, 1)
    h = wid & 1

    cc = pltpu.make_async_copy(tbl.at[f], col_v.at[...], sem_a)
    cc.start()
    ci = pltpu.make_async_copy(
        idx_hbm.at[pl.ds(h * HB, HB)], idx_v.at[...], sem_b
    )
    ci.start()
    ce = pltpu.make_async_copy(
        etbl.at[f, pl.ds(0, VE)], ecol_v.at[...], sem_c
    )
    ce.start()
    cie = pltpu.make_async_copy(
        eidx_hbm.at[pl.ds(h * HB, HB)], eidx_v.at[...], sem_d
    )
    cie.start()

    def _gloop(colref, idxref):
        def _blk(g, _):
            for j in range(8):
                iv = idxref[pl.ds((g * 8 + j) * L, L)]
                out_v[g, pl.ds(j * L, L)] = plsc.load_gather(colref, [iv])
            return 0

        lax.fori_loop(0, NROW, _blk, 0)

    cc.wait()
    ci.wait()
    _gloop(col_v, idx_v)
    co = pltpu.make_async_copy(
        out_v.at[...], piece.at[f, pl.ds(h * NROW, NROW)], sem_a
    )
    co.start()
    co.wait()

    ce.wait()
    cie.wait()
    _gloop(ecol_v, eidx_v)
    co2 = pltpu.make_async_copy(
        out_v.at[...], epiece.at[f, pl.ds(h * NROW, NROW)], sem_a
    )
    co2.start()
    co2.wait()


@functools.partial(
    pl.kernel,
    out_type=jax.ShapeDtypeStruct((D, 128, 128), jnp.float32),
    mesh=_mesh,
    scratch_types=_scratch,
    compiler_params=_params,
)
def _gather_big(idx_hbm, tbl, piece, col_v, idx_v, out_v, sem_a, sem_b):
    wid = lax.axis_index("s") * NC + lax.axis_index("c")
    f = lax.shift_right_logical(wid, 1)
    h = wid & 1
    _one_feature(idx_hbm, tbl, piece, col_v, idx_v, out_v, sem_a, sem_b,
                 f, h, V)


BLK = 4096


def _mlp_body(p0, p1, p2, p3, w1_ref, b1_ref, w2_ref, b2_ref,
              w3_ref, b3_ref, o_ref):
    x = jnp.concatenate(
        [p0[...].reshape(D, BLK), p1[...].reshape(D, BLK),
         p2[...].reshape(D, BLK), p3[...].reshape(D, BLK)], axis=0
    )
    dn = (((0,), (0,)), ((), ()))
    h = lax.dot_general(w1_ref[...], x, dn,
                        preferred_element_type=jnp.float32)
    h = jnp.maximum(h + b1_ref[...], 0.0)
    h2 = lax.dot_general(w2_ref[...], h, dn,
                         preferred_element_type=jnp.float32)
    h2 = jnp.maximum(h2 + b2_ref[...], 0.0)
    z = lax.dot_general(w3_ref[...], h2, dn,
                        preferred_element_type=jnp.float32)
    o_ref[...] = jax.nn.sigmoid(z + b3_ref[0, 0])[0]


def _mlp(pieces, W1, b1, W2, b2, W3, b3):
    grid = (B // BLK,)
    pspec = pl.BlockSpec((D, BLK // 128, 128), lambda i: (0, i, 0))
    whole = lambda shape: pl.BlockSpec(shape, lambda i: (0,) * len(shape))
    return pl.pallas_call(
        _mlp_body,
        grid=grid,
        in_specs=[
            pspec, pspec, pspec, pspec,
            whole((64, 64)), whole((64, 1)),
            whole((64, 32)), whole((32, 1)),
            whole((32, 1)), whole((1, 1)),
        ],
        out_specs=pl.BlockSpec((BLK,), lambda i: (i,)),
        out_shape=jax.ShapeDtypeStruct((B,), jnp.float32),
    )(*pieces, W1, b1, W2, b2, W3, b3)


def kernel(skills, positions, education, job_position,
           skill_table, position_table, education_table, job_position_table,
           W1, b1, W2, b2, W3, b3):
    sk = skills.astype(jnp.int32)
    po = positions.astype(jnp.int32)
    ed = education.astype(jnp.int32)
    jp = job_position.astype(jnp.int32)

    p_sk, p_ed = _gather_big_edu(sk, ed, skill_table.T, education_table.T)
    pieces = [
        p_sk,
        _gather_big(po, position_table.T),
        p_ed,
        _gather_big(jp, job_position_table.T),
    ]

    return _mlp(
        pieces, W1, b1.reshape(64, 1), W2, b2.reshape(32, 1),
        W3, b3.reshape(1, 1),
    )


# speedup vs baseline: 1.0720x; 1.0112x over previous
import functools

import jax
import jax.numpy as jnp
from jax import lax
from jax.experimental import pallas as pl
from jax.experimental.pallas import tpu as pltpu
from jax.experimental.pallas import tpu_sc as plsc

B = 16384
D = 16
V = 100000
VE = 1000
NC = 2
L = 16
HB = B // 2
NROW = HB // 128

_mesh = plsc.VectorSubcoreMesh(core_axis_name="c", subcore_axis_name="s")


def _one_feature(idx_hbm, tbl, piece, col_v, idx_v, out_v, sem_a, sem_b,
                 f, h, vocab):
    cc = pltpu.make_async_copy(
        tbl.at[f, pl.ds(0, vocab)], col_v.at[pl.ds(0, vocab)], sem_a
    )
    cc.start()
    ci = pltpu.make_async_copy(
        idx_hbm.at[pl.ds(h * HB, HB)], idx_v.at[...], sem_b
    )
    ci.start()
    cc.wait()
    ci.wait()

    def _blk(g, _):
        for j in range(8):
            iv = idx_v[pl.ds((g * 8 + j) * L, L)]
            out_v[g, pl.ds(j * L, L)] = plsc.load_gather(col_v, [iv])
        return 0

    lax.fori_loop(0, NROW, _blk, 0)

    co = pltpu.make_async_copy(
        out_v.at[...], piece.at[f, pl.ds(h * NROW, NROW)], sem_a
    )
    co.start()
    co.wait()


_scratch = [
    pltpu.VMEM((V,), jnp.float32),
    pltpu.VMEM((HB,), jnp.int32),
    pltpu.VMEM((NROW, 128), jnp.float32),
    pltpu.SemaphoreType.DMA,
    pltpu.SemaphoreType.DMA,
]
_params = pltpu.CompilerParams(
    use_tc_tiling_on_sc=False, needs_layout_passes=False
)


@functools.partial(
    pl.kernel,
    out_type=[jax.ShapeDtypeStruct((D, 128, 128), jnp.float32)] * 2,
    mesh=_mesh,
    scratch_types=[
        pltpu.VMEM((V,), jnp.float32),
        pltpu.VMEM((HB,), jnp.int32),
        pltpu.VMEM((NROW, 128), jnp.float32),
        pltpu.VMEM((VE,), jnp.float32),
        pltpu.VMEM((HB,), jnp.int32),
        pltpu.SemaphoreType.DMA,
        pltpu.SemaphoreType.DMA,
        pltpu.SemaphoreType.DMA,
        pltpu.SemaphoreType.DMA,
    ],
    compiler_params=_params,
)
def _gather_big_edu(idx_hbm, eidx_hbm, tbl, etbl, piece, epiece,
                    col_v, idx_v, out_v, ecol_v, eidx_v,
                    sem_a, sem_b, sem_c, sem_d):
    wid = lax.axis_index("s") * NC + lax.axis_index("c")
    f = lax.shift_right_logical(wid, 1)
    h = wid & 1

    cc = pltpu.make_async_copy(tbl.at[f], col_v.at[...], sem_a)
    cc.start()
    ci = pltpu.make_async_copy(
        idx_hbm.at[pl.ds(h * HB, HB)], idx_v.at[...], sem_b
    )
    ci.start()
    ce = pltpu.make_async_copy(
        etbl.at[f, pl.ds(0, VE)], ecol_v.at[...], sem_c
    )
    ce.start()
    cie = pltpu.make_async_copy(
        eidx_hbm.at[pl.ds(h * HB, HB)], eidx_v.at[...], sem_d
    )
    cie.start()

    def _gloop(colref, idxref):
        def _blk(g, _):
            for j in range(8):
                iv = idxref[pl.ds((g * 8 + j) * L, L)]
                out_v[g, pl.ds(j * L, L)] = plsc.load_gather(colref, [iv])
            return 0

        lax.fori_loop(0, NROW, _blk, 0)

    cc.wait()
    ci.wait()
    _gloop(col_v, idx_v)
    co = pltpu.make_async_copy(
        out_v.at[...], piece.at[f, pl.ds(h * NROW, NROW)], sem_a
    )
    co.start()
    co.wait()

    ce.wait()
    cie.wait()
    _gloop(ecol_v, eidx_v)
    co2 = pltpu.make_async_copy(
        out_v.at[...], epiece.at[f, pl.ds(h * NROW, NROW)], sem_a
    )
    co2.start()
    co2.wait()


@functools.partial(
    pl.kernel,
    out_type=jax.ShapeDtypeStruct((D, 128, 128), jnp.float32),
    mesh=_mesh,
    scratch_types=_scratch,
    compiler_params=_params,
)
def _gather_big(idx_hbm, tbl, piece, col_v, idx_v, out_v, sem_a, sem_b):
    wid = lax.axis_index("s") * NC + lax.axis_index("c")
    f = lax.shift_right_logical(wid, 1)
    h = wid & 1
    _one_feature(idx_hbm, tbl, piece, col_v, idx_v, out_v, sem_a, sem_b,
                 f, h, V)


BLK = 8192


def _mlp_body(p0, p1, p2, p3, w1_ref, b1_ref, w2_ref, b2_ref,
              w3_ref, b3_ref, o_ref):
    x = jnp.concatenate(
        [p0[...].reshape(D, BLK), p1[...].reshape(D, BLK),
         p2[...].reshape(D, BLK), p3[...].reshape(D, BLK)], axis=0
    )
    dn = (((0,), (0,)), ((), ()))
    h = lax.dot_general(w1_ref[...], x, dn,
                        preferred_element_type=jnp.float32)
    h = jnp.maximum(h + b1_ref[...], 0.0)
    h2 = lax.dot_general(w2_ref[...], h, dn,
                         preferred_element_type=jnp.float32)
    h2 = jnp.maximum(h2 + b2_ref[...], 0.0)
    z = lax.dot_general(w3_ref[...], h2, dn,
                        preferred_element_type=jnp.float32)
    o_ref[...] = jax.nn.sigmoid(z + b3_ref[0, 0])[0]


def _mlp(pieces, W1, b1, W2, b2, W3, b3):
    grid = (B // BLK,)
    pspec = pl.BlockSpec((D, BLK // 128, 128), lambda i: (0, i, 0))
    whole = lambda shape: pl.BlockSpec(shape, lambda i: (0,) * len(shape))
    return pl.pallas_call(
        _mlp_body,
        grid=grid,
        in_specs=[
            pspec, pspec, pspec, pspec,
            whole((64, 64)), whole((64, 1)),
            whole((64, 32)), whole((32, 1)),
            whole((32, 1)), whole((1, 1)),
        ],
        out_specs=pl.BlockSpec((BLK,), lambda i: (i,)),
        out_shape=jax.ShapeDtypeStruct((B,), jnp.float32),
    )(*pieces, W1, b1, W2, b2, W3, b3)


def kernel(skills, positions, education, job_position,
           skill_table, position_table, education_table, job_position_table,
           W1, b1, W2, b2, W3, b3):
    sk = skills.astype(jnp.int32)
    po = positions.astype(jnp.int32)
    ed = education.astype(jnp.int32)
    jp = job_position.astype(jnp.int32)

    p_sk, p_ed = _gather_big_edu(sk, ed, skill_table.T, education_table.T)
    pieces = [
        p_sk,
        _gather_big(po, position_table.T),
        p_ed,
        _gather_big(jp, job_position_table.T),
    ]

    return _mlp(
        pieces, W1, b1.reshape(64, 1), W2, b2.reshape(32, 1),
        W3, b3.reshape(1, 1),
    )
